# baseline (speedup 1.0000x reference)
import functools

import jax
import jax.numpy as jnp
from jax import lax
from jax.experimental import pallas as pl
from jax.experimental.pallas import tpu as pltpu
from jax.experimental.pallas import tpu_sc as plsc

Q = 8
D = 768
R = 4
KCOEF = 0.1
_BIG_I32 = 2147483647

NW = 32
RW = 1280
CH = 32
RB = 4
SC_ROWS = NW * RW
KT = 100000 - SC_ROWS
NBH = 15
BK = KT // (2 * NBH)


def _block_top4(qn, x, ml, lf, base):
    rawT = lax.dot_general(
        x, qn, (((1,), (1,)), ((), ())),
        preferred_element_type=jnp.float32,
    )
    xss = jnp.sum(x * x, axis=1, keepdims=True)
    sem = rawT * lax.rsqrt(jnp.maximum(xss, 1e-16))

    rel = jnp.abs(ml - lf) / jnp.maximum(ml, lf)
    s = sem * jnp.exp(-rel * KCOEF)

    giota = lax.broadcasted_iota(jnp.int32, (BK, Q), 0) + base
    bvals, bidx = [], []
    for _ in range(R):
        m = jnp.max(s, axis=0, keepdims=True)
        am = jnp.min(jnp.where(s == m, giota, _BIG_I32), axis=0,
                     keepdims=True)
        bvals.append(m)
        bidx.append(am)
        s = jnp.where(giota == am, -jnp.inf, s)
    return bvals, bidx


def _score_topk_body(q_ref, xa_ref, xb_ref, mla_ref, mlb_ref, lf_ref,
                     vals_ref, idx_ref):
    i = pl.program_id(0)

    @pl.when(i == 0)
    def _init():
        vals_ref[...] = jnp.full((R, Q), -jnp.inf, jnp.float32)
        idx_ref[...] = jnp.zeros((R, Q), jnp.int32)

    q = q_ref[...]
    lf = lf_ref[...]

    qss = jnp.sum(q * q, axis=1, keepdims=True)
    qn = q * lax.rsqrt(jnp.maximum(qss, 1e-16))

    av, ai = _block_top4(qn, xa_ref[...], mla_ref[...], lf, i * BK)
    bv, bi = _block_top4(qn, xb_ref[...], mlb_ref[...], lf, (i + NBH) * BK)

    cat_v = jnp.concatenate([vals_ref[...]] + av + bv, axis=0)
    cat_i = jnp.concatenate([idx_ref[...]] + ai + bi, axis=0)
    pos = lax.broadcasted_iota(jnp.int32, (3 * R, Q), 0)
    nv, ni = [], []
    for _ in range(R):
        m = jnp.max(cat_v, axis=0, keepdims=True)
        p = jnp.min(jnp.where(cat_v == m, pos, _BIG_I32), axis=0,
                    keepdims=True)
        sel = pos == p
        nv.append(m)
        ni.append(jnp.sum(jnp.where(sel, cat_i, 0), axis=0, keepdims=True))
        cat_v = jnp.where(sel, -jnp.inf, cat_v)
    vals_ref[...] = jnp.concatenate(nv, axis=0)
    idx_ref[...] = jnp.concatenate(ni, axis=0)


def _score_topk(query, x, ml2, lf, interpret=False):
    return pl.pallas_call(
        _score_topk_body,
        grid=(NBH,),
        in_specs=[
            pl.BlockSpec((Q, D), lambda i: (0, 0)),
            pl.BlockSpec((BK, D), lambda i: (i, 0)),
            pl.BlockSpec((BK, D), lambda i: (i + NBH, 0)),
            pl.BlockSpec((BK, 1), lambda i: (i, 0)),
            pl.BlockSpec((BK, 1), lambda i: (i + NBH, 0)),
            pl.BlockSpec((1, Q), lambda i: (0, 0)),
        ],
        out_specs=[
            pl.BlockSpec((R, Q), lambda i: (0, 0)),
            pl.BlockSpec((R, Q), lambda i: (0, 0)),
        ],
        out_shape=[
            jax.ShapeDtypeStruct((R, Q), jnp.float32),
            jax.ShapeDtypeStruct((R, Q), jnp.int32),
        ],
        interpret=interpret,
    )(query, x, x, ml2, ml2, lf)


@functools.lru_cache(maxsize=1)
def _build_sc_score():
    mesh = plsc.VectorSubcoreMesh(core_axis_name="c", subcore_axis_name="s")

    @functools.partial(
        pl.kernel,
        mesh=mesh,
        out_type=[
            jax.ShapeDtypeStruct((NW * R, 16), jnp.float32),
            jax.ShapeDtypeStruct((NW * R, 16), jnp.float32),
            jax.ShapeDtypeStruct((NW * R, 16), jnp.int32),
        ],
        scratch_types=[
            pltpu.VMEM((Q, D), jnp.float32),
            pltpu.VMEM((CH, D), jnp.float32),
            pltpu.VMEM((CH, D), jnp.float32),
            pltpu.VMEM((CH, 16), jnp.float32),
            pltpu.VMEM((CH, 16), jnp.float32),
            pltpu.VMEM((16,), jnp.float32),
            pltpu.VMEM((R, 16), jnp.float32),
            pltpu.VMEM((R, 16), jnp.float32),
            pltpu.VMEM((R, 16), jnp.int32),
            pltpu.SemaphoreType.DMA,
            pltpu.SemaphoreType.DMA,
            pltpu.SemaphoreType.DMA,
            pltpu.SemaphoreType.DMA,
        ],
    )
    def sc_score(q_hbm, x_hbm, ml_hbm, lf_hbm, ork_hbm, oxs_hbm, oix_hbm,
                 qb, xb0, xb1, mb0, mb1, lfb, srk, sxs, six,
                 sem0, sem1, sem2, sem3):
        w = lax.axis_index("s") * 2 + lax.axis_index("c")
        row0 = KT + w * RW

        pltpu.sync_copy(q_hbm, qb)
        pltpu.sync_copy(lf_hbm, lfb)
        lfv = lfb[...]
        lane = lax.iota(jnp.int32, 16)

        gdn = lax.GatherDimensionNumbers(
            offset_dims=(), collapsed_slice_dims=(0,), start_index_map=(0,))

        def lperm(v, idx):
            return lax.gather(v, idx[:, None], gdn, slice_sizes=(1,),
                              mode=lax.GatherScatterMode.PROMISE_IN_BOUNDS)

        def hsum(v):
            for m in (8, 4, 2, 1):
                v = v + lperm(v, lane ^ m)
            return v

        xbufs, mbufs, xsems, msems = (xb0, xb1), (mb0, mb1), (sem0, sem1), (sem2, sem3)

        def start_dma(c, b):
            st = row0 + c * CH
            pltpu.make_async_copy(x_hbm.at[pl.ds(st, CH)], xbufs[b], xsems[b]).start()
            pltpu.make_async_copy(ml_hbm.at[pl.ds(st, CH)], mbufs[b], msems[b]).start()

        def wait_dma(c, b):
            st = row0 + c * CH
            pltpu.make_async_copy(x_hbm.at[pl.ds(st, CH)], xbufs[b], xsems[b]).wait()
            pltpu.make_async_copy(ml_hbm.at[pl.ds(st, CH)], mbufs[b], msems[b]).wait()

        start_dma(0, 0)

        def chunk(c, b, carry):
            nxt = c + 1

            @pl.when(nxt < RW // CH)
            def _():
                start_dma(nxt, 1 - b)

            wait_dma(c, b)
            xb, mb = xbufs[b], mbufs[b]

            def group(g, carry):
                accs = [[jnp.zeros((16,), jnp.float32) for _ in range(Q + 1)]
                        for _ in range(RB)]
                for j in range(D // 16):
                    qv = [qb[qq, pl.ds(j * 16, 16)] for qq in range(Q)]
                    for rr in range(RB):
                        xv = xb[g * RB + rr, pl.ds(j * 16, 16)]
                        for qq in range(Q):
                            accs[rr][qq] = accs[rr][qq] + xv * qv[qq]
                        accs[rr][Q] = accs[rr][Q] + xv * xv
                bk0, bk1, bk2, bk3, bi0, bi1, bi2, bi3, \
                    br0, br1, br2, br3, bx0, bx1, bx2, bx3 = carry
                bks = [bk0, bk1, bk2, bk3]
                bis = [bi0, bi1, bi2, bi3]
                brs = [br0, br1, br2, br3]
                bxs = [bx0, bx1, bx2, bx3]
                for rr in range(RB):
                    rawv = jnp.zeros((16,), jnp.float32)
                    for qq in range(Q):
                        rawv = jnp.where(lane == qq, hsum(accs[rr][qq]), rawv)
                    xssv = hsum(accs[rr][Q])
                    mlv = mb[g * RB + rr, pl.ds(0, 16)]
                    relv = jnp.abs(mlv - lfv) / jnp.maximum(mlv, lfv)
                    rkv = rawv * jnp.exp(relv * (-KCOEF))
                    keyv = rkv * jnp.abs(rkv) / jnp.maximum(xssv, 1e-16)
                    riv = lax.broadcast(row0 + c * CH + g * RB + rr, (16,))
                    vk, vi, vr, vx = keyv, riv, rkv, xssv
                    for k in range(R):
                        swap = vk > bks[k]
                        bks[k], vk = (jnp.where(swap, vk, bks[k]),
                                      jnp.where(swap, bks[k], vk))
                        bis[k], vi = (jnp.where(swap, vi, bis[k]),
                                      jnp.where(swap, bis[k], vi))
                        brs[k], vr = (jnp.where(swap, vr, brs[k]),
                                      jnp.where(swap, brs[k], vr))
                        bxs[k], vx = (jnp.where(swap, vx, bxs[k]),
                                      jnp.where(swap, bxs[k], vx))
                return tuple(bks) + tuple(bis) + tuple(brs) + tuple(bxs)

            return lax.fori_loop(0, CH // RB, group, carry)

        carry = tuple([jnp.full((16,), -jnp.inf, jnp.float32)] * R
                      + [jnp.zeros((16,), jnp.int32)] * R
                      + [jnp.zeros((16,), jnp.float32)] * R
                      + [jnp.ones((16,), jnp.float32)] * R)

        def twochunks(g2, carry):
            carry = chunk(g2 * 2, 0, carry)
            carry = chunk(g2 * 2 + 1, 1, carry)
            return carry

        carry = lax.fori_loop(0, RW // (2 * CH), twochunks, carry)

        for k in range(R):
            srk[k, pl.ds(0, 16)] = carry[2 * R + k]
            sxs[k, pl.ds(0, 16)] = carry[3 * R + k]
            six[k, pl.ds(0, 16)] = carry[R + k]
        pltpu.sync_copy(srk, ork_hbm.at[pl.ds(w * R, R)])
        pltpu.sync_copy(sxs, oxs_hbm.at[pl.ds(w * R, R)])
        pltpu.sync_copy(six, oix_hbm.at[pl.ds(w * R, R)])

    return sc_score


def _merge_body(tcv_ref, tci_ref, srk_ref, sxs_ref, six_ref, qss_ref,
                vals_ref, idx_ref):
    qss = qss_ref[...]
    cq = jnp.maximum(jnp.sqrt(qss), 1e-8)
    cq16 = jnp.concatenate([cq, jnp.ones((1, 16 - Q), jnp.float32)], axis=1)

    srk = srk_ref[...]
    sxs = sxs_ref[...]
    six = six_ref[...]
    sc_s = srk * lax.rsqrt(jnp.maximum(sxs, 1e-16)) / cq16
    lane = lax.broadcasted_iota(jnp.int32, (NW * R, 16), 1)
    sc_s = jnp.where(lane < Q, sc_s, -jnp.inf)

    pad_v = jnp.full((R, 16 - Q), -jnp.inf, jnp.float32)
    pad_i = jnp.zeros((R, 16 - Q), jnp.int32)
    cat_v = jnp.concatenate(
        [jnp.concatenate([tcv_ref[...], pad_v], axis=1), sc_s], axis=0)
    cat_i = jnp.concatenate(
        [jnp.concatenate([tci_ref[...], pad_i], axis=1), six], axis=0)

    n = NW * R + R
    pos = lax.broadcasted_iota(jnp.int32, (n, 16), 0)
    nv, ni = [], []
    for _ in range(R):
        m = jnp.max(cat_v, axis=0, keepdims=True)
        p = jnp.min(jnp.where(cat_v == m, pos, _BIG_I32), axis=0,
                    keepdims=True)
        sel = pos == p
        nv.append(m)
        ni.append(jnp.sum(jnp.where(sel, cat_i, 0), axis=0, keepdims=True))
        cat_v = jnp.where(sel, -jnp.inf, cat_v)
    vals_ref[...] = jnp.concatenate(nv, axis=0)[:, :Q]
    idx_ref[...] = jnp.concatenate(ni, axis=0)[:, :Q]


def _merge(tcv, tci, srk, sxs, six, qss, interpret=False):
    return pl.pallas_call(
        _merge_body,
        out_shape=[
            jax.ShapeDtypeStruct((R, Q), jnp.float32),
            jax.ShapeDtypeStruct((R, Q), jnp.int32),
        ],
        interpret=interpret,
    )(tcv, tci, srk, sxs, six, qss)


@functools.lru_cache(maxsize=1)
def _build_gather_weight():
    mesh = plsc.VectorSubcoreMesh(core_axis_name="c", subcore_axis_name="s")

    @functools.partial(
        pl.kernel,
        mesh=mesh,
        out_type=jax.ShapeDtypeStruct((Q * R, D), jnp.float32),
        scratch_types=[
            pltpu.VMEM((Q * R,), jnp.int32),
            pltpu.VMEM((16,), jnp.float32),
            pltpu.VMEM((Q * R, D), jnp.float32),
            pltpu.VMEM((D,), jnp.float32),
            pltpu.SemaphoreType.DMA,
        ],
    )
    def gather_weight(idx_hbm, scb_hbm, table_hbm, out_hbm,
                      idx_v, scb_v, rows_v, out_v, sem):
        w = lax.axis_index("s") * 2 + lax.axis_index("c")
        pltpu.sync_copy(idx_hbm, idx_v)
        pltpu.async_copy(table_hbm.at[idx_v], rows_v, sem).wait()
        pltpu.sync_copy(scb_hbm.at[w], scb_v)
        score = scb_v[...]
        for j in range(D // 16):
            out_v[pl.ds(j * 16, 16)] = rows_v[w, pl.ds(j * 16, 16)] * score
        pltpu.sync_copy(out_v, out_hbm.at[w])

    return gather_weight


def kernel(query, text_features, lengths, motion_lengths):
    k = text_features.shape[0]
    lf = lengths.astype(jnp.float32).reshape(1, Q)
    lf16 = jnp.concatenate([lf, jnp.ones((1, 16 - Q), jnp.float32)],
                           axis=1).reshape(16)
    ml2 = motion_lengths.astype(jnp.float32).reshape(k, 1)
    mlb = jnp.broadcast_to(ml2, (k, 16))

    srk, sxs, six = _build_sc_score()(query, text_features, mlb, lf16)
    tcv, tci = _score_topk(query, text_features, ml2, lf)

    qss = jnp.sum(query * query, axis=1).reshape(1, Q)
    vals_t, idx_t = _merge(tcv, tci, srk, sxs, six, qss)

    top_vals = vals_t.T
    top_idx = idx_t.T
    score_bcast = jnp.broadcast_to(top_vals.reshape(Q * R, 1), (Q * R, 16))
    weighted = _build_gather_weight()(top_idx.reshape(Q * R),
                                      score_bcast,
                                      text_features)
    return weighted.reshape(Q, R, D), top_idx


# speedup vs baseline: 3.7492x; 3.7492x over previous
import functools

import jax
import jax.numpy as jnp
from jax import lax
from jax.experimental import pallas as pl
from jax.experimental.pallas import tpu as pltpu
from jax.experimental.pallas import tpu_sc as plsc

Q = 8
D = 768
R = 4
KCOEF = 0.1
_BIG_I32 = 2147483647

NW = 32
RW = 512
CH = 32
RB = 2
SC_ROWS = NW * RW
KT = 100000 - SC_ROWS
NBH = 26
BK = KT // (2 * NBH)


def _block_top4(qn, x, ml, lf, base):
    rawT = lax.dot_general(
        x, qn, (((1,), (1,)), ((), ())),
        preferred_element_type=jnp.float32,
    )
    xss = jnp.sum(x * x, axis=1, keepdims=True)
    sem = rawT * lax.rsqrt(jnp.maximum(xss, 1e-16))

    rel = jnp.abs(ml - lf) / jnp.maximum(ml, lf)
    s = sem * jnp.exp(-rel * KCOEF)

    giota = lax.broadcasted_iota(jnp.int32, (BK, Q), 0) + base
    bvals, bidx = [], []
    for _ in range(R):
        m = jnp.max(s, axis=0, keepdims=True)
        am = jnp.min(jnp.where(s == m, giota, _BIG_I32), axis=0,
                     keepdims=True)
        bvals.append(m)
        bidx.append(am)
        s = jnp.where(giota == am, -jnp.inf, s)
    return bvals, bidx


def _score_topk_body(q_ref, xa_ref, xb_ref, mla_ref, mlb_ref, lf_ref,
                     vals_ref, idx_ref):
    i = pl.program_id(0)

    @pl.when(i == 0)
    def _init():
        vals_ref[...] = jnp.full((R, Q), -jnp.inf, jnp.float32)
        idx_ref[...] = jnp.zeros((R, Q), jnp.int32)

    q = q_ref[...]
    lf = lf_ref[...]

    qss = jnp.sum(q * q, axis=1, keepdims=True)
    qn = q * lax.rsqrt(jnp.maximum(qss, 1e-16))

    av, ai = _block_top4(qn, xa_ref[...], mla_ref[...], lf, i * BK)
    bv, bi = _block_top4(qn, xb_ref[...], mlb_ref[...], lf, (i + NBH) * BK)

    cat_v = jnp.concatenate([vals_ref[...]] + av + bv, axis=0)
    cat_i = jnp.concatenate([idx_ref[...]] + ai + bi, axis=0)
    pos = lax.broadcasted_iota(jnp.int32, (3 * R, Q), 0)
    nv, ni = [], []
    for _ in range(R):
        m = jnp.max(cat_v, axis=0, keepdims=True)
        p = jnp.min(jnp.where(cat_v == m, pos, _BIG_I32), axis=0,
                    keepdims=True)
        sel = pos == p
        nv.append(m)
        ni.append(jnp.sum(jnp.where(sel, cat_i, 0), axis=0, keepdims=True))
        cat_v = jnp.where(sel, -jnp.inf, cat_v)
    vals_ref[...] = jnp.concatenate(nv, axis=0)
    idx_ref[...] = jnp.concatenate(ni, axis=0)


def _score_topk(query, x, ml2, lf, interpret=False):
    return pl.pallas_call(
        _score_topk_body,
        grid=(NBH,),
        in_specs=[
            pl.BlockSpec((Q, D), lambda i: (0, 0)),
            pl.BlockSpec((BK, D), lambda i: (i, 0)),
            pl.BlockSpec((BK, D), lambda i: (i + NBH, 0)),
            pl.BlockSpec((BK, 1), lambda i: (i, 0)),
            pl.BlockSpec((BK, 1), lambda i: (i + NBH, 0)),
            pl.BlockSpec((1, Q), lambda i: (0, 0)),
        ],
        out_specs=[
            pl.BlockSpec((R, Q), lambda i: (0, 0)),
            pl.BlockSpec((R, Q), lambda i: (0, 0)),
        ],
        out_shape=[
            jax.ShapeDtypeStruct((R, Q), jnp.float32),
            jax.ShapeDtypeStruct((R, Q), jnp.int32),
        ],
        interpret=interpret,
    )(query, x, x, ml2, ml2, lf)


@functools.lru_cache(maxsize=1)
def _build_sc_score():
    mesh = plsc.VectorSubcoreMesh(core_axis_name="c", subcore_axis_name="s")

    @functools.partial(
        pl.kernel,
        mesh=mesh,
        out_type=[
            jax.ShapeDtypeStruct((NW * R, 16), jnp.float32),
            jax.ShapeDtypeStruct((NW * R, 16), jnp.float32),
            jax.ShapeDtypeStruct((NW * R, 16), jnp.int32),
        ],
        scratch_types=[
            pltpu.VMEM((Q, D), jnp.float32),
            pltpu.VMEM((CH, D), jnp.float32),
            pltpu.VMEM((CH, D), jnp.float32),
            pltpu.VMEM((CH, 16), jnp.float32),
            pltpu.VMEM((CH, 16), jnp.float32),
            pltpu.VMEM((16,), jnp.float32),
            pltpu.VMEM((R, 16), jnp.float32),
            pltpu.VMEM((R, 16), jnp.float32),
            pltpu.VMEM((R, 16), jnp.int32),
            pltpu.SemaphoreType.DMA,
            pltpu.SemaphoreType.DMA,
            pltpu.SemaphoreType.DMA,
            pltpu.SemaphoreType.DMA,
        ],
    )
    def sc_score(q_hbm, x_hbm, ml_hbm, lf_hbm, ork_hbm, oxs_hbm, oix_hbm,
                 qb, xb0, xb1, mb0, mb1, lfb, srk, sxs, six,
                 sem0, sem1, sem2, sem3):
        w = lax.axis_index("s") * 2 + lax.axis_index("c")
        row0 = KT + w * RW

        pltpu.sync_copy(q_hbm, qb)
        pltpu.sync_copy(lf_hbm, lfb)
        lfv = lfb[...]
        lane = lax.iota(jnp.int32, 16)

        gdn = lax.GatherDimensionNumbers(
            offset_dims=(), collapsed_slice_dims=(0,), start_index_map=(0,))

        def lperm(v, idx):
            return lax.gather(v, idx[:, None], gdn, slice_sizes=(1,),
                              mode=lax.GatherScatterMode.PROMISE_IN_BOUNDS)

        def hsum(v):
            for m in (8, 4, 2, 1):
                v = v + lperm(v, lane ^ m)
            return v

        xbufs, mbufs, xsems, msems = (xb0, xb1), (mb0, mb1), (sem0, sem1), (sem2, sem3)

        def start_dma(c, b):
            st = row0 + c * CH
            pltpu.make_async_copy(x_hbm.at[pl.ds(st, CH)], xbufs[b], xsems[b]).start()
            pltpu.make_async_copy(ml_hbm.at[pl.ds(st, CH)], mbufs[b], msems[b]).start()

        def wait_dma(c, b):
            st = row0 + c * CH
            pltpu.make_async_copy(x_hbm.at[pl.ds(st, CH)], xbufs[b], xsems[b]).wait()
            pltpu.make_async_copy(ml_hbm.at[pl.ds(st, CH)], mbufs[b], msems[b]).wait()

        start_dma(0, 0)

        def chunk(c, b, carry):
            nxt = c + 1

            @pl.when(nxt < RW // CH)
            def _():
                start_dma(nxt, 1 - b)

            wait_dma(c, b)
            xb, mb = xbufs[b], mbufs[b]

            def group(g, carry):
                accs = [[jnp.zeros((16,), jnp.float32) for _ in range(Q + 1)]
                        for _ in range(RB)]
                for j in range(D // 16):
                    qv = [qb[qq, pl.ds(j * 16, 16)] for qq in range(Q)]
                    for rr in range(RB):
                        xv = xb[g * RB + rr, pl.ds(j * 16, 16)]
                        for qq in range(Q):
                            accs[rr][qq] = accs[rr][qq] + xv * qv[qq]
                        accs[rr][Q] = accs[rr][Q] + xv * xv
                bk0, bk1, bk2, bk3, bi0, bi1, bi2, bi3, \
                    br0, br1, br2, br3, bx0, bx1, bx2, bx3 = carry
                bks = [bk0, bk1, bk2, bk3]
                bis = [bi0, bi1, bi2, bi3]
                brs = [br0, br1, br2, br3]
                bxs = [bx0, bx1, bx2, bx3]
                for rr in range(RB):
                    rawv = jnp.zeros((16,), jnp.float32)
                    for qq in range(Q):
                        rawv = jnp.where(lane == qq, hsum(accs[rr][qq]), rawv)
                    xssv = hsum(accs[rr][Q])
                    mlv = mb[g * RB + rr, pl.ds(0, 16)]
                    relv = jnp.abs(mlv - lfv) / jnp.maximum(mlv, lfv)
                    rkv = rawv * jnp.exp(relv * (-KCOEF))
                    keyv = rkv * jnp.abs(rkv) / jnp.maximum(xssv, 1e-16)
                    riv = lax.broadcast(row0 + c * CH + g * RB + rr, (16,))
                    vk, vi, vr, vx = keyv, riv, rkv, xssv
                    for k in range(R):
                        swap = vk > bks[k]
                        bks[k], vk = (jnp.where(swap, vk, bks[k]),
                                      jnp.where(swap, bks[k], vk))
                        bis[k], vi = (jnp.where(swap, vi, bis[k]),
                                      jnp.where(swap, bis[k], vi))
                        brs[k], vr = (jnp.where(swap, vr, brs[k]),
                                      jnp.where(swap, brs[k], vr))
                        bxs[k], vx = (jnp.where(swap, vx, bxs[k]),
                                      jnp.where(swap, bxs[k], vx))
                return tuple(bks) + tuple(bis) + tuple(brs) + tuple(bxs)

            return lax.fori_loop(0, CH // RB, group, carry)

        carry = tuple([jnp.full((16,), -jnp.inf, jnp.float32)] * R
                      + [jnp.zeros((16,), jnp.int32)] * R
                      + [jnp.zeros((16,), jnp.float32)] * R
                      + [jnp.ones((16,), jnp.float32)] * R)

        def twochunks(g2, carry):
            carry = chunk(g2 * 2, 0, carry)
            carry = chunk(g2 * 2 + 1, 1, carry)
            return carry

        carry = lax.fori_loop(0, RW // (2 * CH), twochunks, carry)

        for k in range(R):
            srk[k, pl.ds(0, 16)] = carry[2 * R + k]
            sxs[k, pl.ds(0, 16)] = carry[3 * R + k]
            six[k, pl.ds(0, 16)] = carry[R + k]
        pltpu.sync_copy(srk, ork_hbm.at[pl.ds(w * R, R)])
        pltpu.sync_copy(sxs, oxs_hbm.at[pl.ds(w * R, R)])
        pltpu.sync_copy(six, oix_hbm.at[pl.ds(w * R, R)])

    return sc_score


def _merge_body(tcv_ref, tci_ref, srk_ref, sxs_ref, six_ref, qss_ref,
                vals_ref, idx_ref):
    qss = qss_ref[...]
    cq = jnp.maximum(jnp.sqrt(qss), 1e-8)
    cq16 = jnp.concatenate([cq, jnp.ones((1, 16 - Q), jnp.float32)], axis=1)

    srk = srk_ref[...]
    sxs = sxs_ref[...]
    six = six_ref[...]
    sc_s = srk * lax.rsqrt(jnp.maximum(sxs, 1e-16)) / cq16
    lane = lax.broadcasted_iota(jnp.int32, (NW * R, 16), 1)
    sc_s = jnp.where(lane < Q, sc_s, -jnp.inf)

    pad_v = jnp.full((R, 16 - Q), -jnp.inf, jnp.float32)
    pad_i = jnp.zeros((R, 16 - Q), jnp.int32)
    cat_v = jnp.concatenate(
        [jnp.concatenate([tcv_ref[...], pad_v], axis=1), sc_s], axis=0)
    cat_i = jnp.concatenate(
        [jnp.concatenate([tci_ref[...], pad_i], axis=1), six], axis=0)

    n = NW * R + R
    pos = lax.broadcasted_iota(jnp.int32, (n, 16), 0)
    nv, ni = [], []
    for _ in range(R):
        m = jnp.max(cat_v, axis=0, keepdims=True)
        p = jnp.min(jnp.where(cat_v == m, pos, _BIG_I32), axis=0,
                    keepdims=True)
        sel = pos == p
        nv.append(m)
        ni.append(jnp.sum(jnp.where(sel, cat_i, 0), axis=0, keepdims=True))
        cat_v = jnp.where(sel, -jnp.inf, cat_v)
    vals_ref[...] = jnp.concatenate(nv, axis=0)[:, :Q]
    idx_ref[...] = jnp.concatenate(ni, axis=0)[:, :Q]


def _merge(tcv, tci, srk, sxs, six, qss, interpret=False):
    return pl.pallas_call(
        _merge_body,
        out_shape=[
            jax.ShapeDtypeStruct((R, Q), jnp.float32),
            jax.ShapeDtypeStruct((R, Q), jnp.int32),
        ],
        interpret=interpret,
    )(tcv, tci, srk, sxs, six, qss)


@functools.lru_cache(maxsize=1)
def _build_gather_weight():
    mesh = plsc.VectorSubcoreMesh(core_axis_name="c", subcore_axis_name="s")

    @functools.partial(
        pl.kernel,
        mesh=mesh,
        out_type=jax.ShapeDtypeStruct((Q * R, D), jnp.float32),
        scratch_types=[
            pltpu.VMEM((Q * R,), jnp.int32),
            pltpu.VMEM((16,), jnp.float32),
            pltpu.VMEM((Q * R, D), jnp.float32),
            pltpu.VMEM((D,), jnp.float32),
            pltpu.SemaphoreType.DMA,
        ],
    )
    def gather_weight(idx_hbm, scb_hbm, table_hbm, out_hbm,
                      idx_v, scb_v, rows_v, out_v, sem):
        w = lax.axis_index("s") * 2 + lax.axis_index("c")
        pltpu.sync_copy(idx_hbm, idx_v)
        pltpu.async_copy(table_hbm.at[idx_v], rows_v, sem).wait()
        pltpu.sync_copy(scb_hbm.at[w], scb_v)
        score = scb_v[...]
        for j in range(D // 16):
            out_v[pl.ds(j * 16, 16)] = rows_v[w, pl.ds(j * 16, 16)] * score
        pltpu.sync_copy(out_v, out_hbm.at[w])

    return gather_weight


def kernel(query, text_features, lengths, motion_lengths):
    k = text_features.shape[0]
    lf = lengths.astype(jnp.float32).reshape(1, Q)
    lf16 = jnp.concatenate([lf, jnp.ones((1, 16 - Q), jnp.float32)],
                           axis=1).reshape(16)
    ml2 = motion_lengths.astype(jnp.float32).reshape(k, 1)
    mlb = jnp.broadcast_to(ml2, (k, 16))

    srk, sxs, six = _build_sc_score()(query, text_features, mlb, lf16)
    tcv, tci = _score_topk(query, text_features, ml2, lf)

    qss = jnp.sum(query * query, axis=1).reshape(1, Q)
    vals_t, idx_t = _merge(tcv, tci, srk, sxs, six, qss)

    top_vals = vals_t.T
    top_idx = idx_t.T
    score_bcast = jnp.broadcast_to(top_vals.reshape(Q * R, 1), (Q * R, 16))
    weighted = _build_gather_weight()(top_idx.reshape(Q * R),
                                      score_bcast,
                                      text_features)
    return weighted.reshape(Q, R, D), top_idx


# speedup vs baseline: 4.2570x; 1.1354x over previous
import functools

import jax
import jax.numpy as jnp
from jax import lax
from jax.experimental import pallas as pl
from jax.experimental.pallas import tpu as pltpu
from jax.experimental.pallas import tpu_sc as plsc

Q = 8
D = 768
R = 4
KCOEF = 0.1
BK = 2000
NBH = 25
_BIG_I32 = 2147483647


def _block_top4(qn, x, ml, lf, base):
    rawT = lax.dot_general(
        x, qn, (((1,), (1,)), ((), ())),
        preferred_element_type=jnp.float32,
    )
    xss = jnp.sum(x * x, axis=1, keepdims=True)
    sem = rawT * lax.rsqrt(jnp.maximum(xss, 1e-16))

    rel = jnp.abs(ml - lf) / jnp.maximum(ml, lf)
    s = sem * jnp.exp(-rel * KCOEF)

    giota = lax.broadcasted_iota(jnp.int32, (BK, Q), 0) + base
    bvals, bidx = [], []
    for _ in range(R):
        m = jnp.max(s, axis=0, keepdims=True)
        am = jnp.min(jnp.where(s == m, giota, _BIG_I32), axis=0,
                     keepdims=True)
        bvals.append(m)
        bidx.append(am)
        s = jnp.where(giota == am, -jnp.inf, s)
    return bvals, bidx


def _score_topk_body(q_ref, xa_ref, xb_ref, mla_ref, mlb_ref, lf_ref,
                     vals_ref, idx_ref):
    i = pl.program_id(0)

    @pl.when(i == 0)
    def _init():
        vals_ref[...] = jnp.full((R, Q), -jnp.inf, jnp.float32)
        idx_ref[...] = jnp.zeros((R, Q), jnp.int32)

    q = q_ref[...]
    lf = lf_ref[...]

    qss = jnp.sum(q * q, axis=1, keepdims=True)
    qn = q * lax.rsqrt(jnp.maximum(qss, 1e-16))

    av, ai = _block_top4(qn, xa_ref[...], mla_ref[...], lf, i * BK)
    bv, bi = _block_top4(qn, xb_ref[...], mlb_ref[...], lf, (i + NBH) * BK)

    cat_v = jnp.concatenate([vals_ref[...]] + av + bv, axis=0)
    cat_i = jnp.concatenate([idx_ref[...]] + ai + bi, axis=0)
    pos = lax.broadcasted_iota(jnp.int32, (3 * R, Q), 0)
    nv, ni = [], []
    for _ in range(R):
        m = jnp.max(cat_v, axis=0, keepdims=True)
        p = jnp.min(jnp.where(cat_v == m, pos, _BIG_I32), axis=0,
                    keepdims=True)
        sel = pos == p
        nv.append(m)
        ni.append(jnp.sum(jnp.where(sel, cat_i, 0), axis=0, keepdims=True))
        cat_v = jnp.where(sel, -jnp.inf, cat_v)
    vals_ref[...] = jnp.concatenate(nv, axis=0)
    idx_ref[...] = jnp.concatenate(ni, axis=0)


def _score_topk(query, x, ml2, lf, interpret=False):
    return pl.pallas_call(
        _score_topk_body,
        grid=(NBH,),
        in_specs=[
            pl.BlockSpec((Q, D), lambda i: (0, 0)),
            pl.BlockSpec((BK, D), lambda i: (i, 0)),
            pl.BlockSpec((BK, D), lambda i: (i + NBH, 0)),
            pl.BlockSpec((BK, 1), lambda i: (i, 0)),
            pl.BlockSpec((BK, 1), lambda i: (i + NBH, 0)),
            pl.BlockSpec((1, Q), lambda i: (0, 0)),
        ],
        out_specs=[
            pl.BlockSpec((R, Q), lambda i: (0, 0)),
            pl.BlockSpec((R, Q), lambda i: (0, 0)),
        ],
        out_shape=[
            jax.ShapeDtypeStruct((R, Q), jnp.float32),
            jax.ShapeDtypeStruct((R, Q), jnp.int32),
        ],
        interpret=interpret,
    )(query, x, x, ml2, ml2, lf)


@functools.lru_cache(maxsize=1)
def _build_gather_weight():
    mesh = plsc.VectorSubcoreMesh(core_axis_name="c", subcore_axis_name="s")

    @functools.partial(
        pl.kernel,
        mesh=mesh,
        out_type=jax.ShapeDtypeStruct((Q * R, D), jnp.float32),
        scratch_types=[
            pltpu.VMEM((Q * R,), jnp.int32),
            pltpu.VMEM((16,), jnp.float32),
            pltpu.VMEM((Q * R, D), jnp.float32),
            pltpu.VMEM((D,), jnp.float32),
            pltpu.SemaphoreType.DMA,
        ],
    )
    def gather_weight(idx_hbm, scb_hbm, table_hbm, out_hbm,
                      idx_v, scb_v, rows_v, out_v, sem):
        w = lax.axis_index("s") * 2 + lax.axis_index("c")
        pltpu.sync_copy(idx_hbm, idx_v)
        pltpu.async_copy(table_hbm.at[idx_v], rows_v, sem).wait()
        pltpu.sync_copy(scb_hbm.at[w], scb_v)
        score = scb_v[...]
        for j in range(D // 16):
            out_v[pl.ds(j * 16, 16)] = rows_v[w, pl.ds(j * 16, 16)] * score
        pltpu.sync_copy(out_v, out_hbm.at[w])

    return gather_weight


def kernel(query, text_features, lengths, motion_lengths):
    k = text_features.shape[0]
    lf = lengths.astype(jnp.float32).reshape(1, Q)
    ml2 = motion_lengths.astype(jnp.float32).reshape(k, 1)
    vals_t, idx_t = _score_topk(query, text_features, ml2, lf)
    top_vals = vals_t.T
    top_idx = idx_t.T
    score_bcast = jnp.broadcast_to(top_vals.reshape(Q * R, 1), (Q * R, 16))
    weighted = _build_gather_weight()(top_idx.reshape(Q * R),
                                      score_bcast,
                                      text_features)
    return weighted.reshape(Q, R, D), top_idx
